# baseline (speedup 1.0000x reference)
import functools

import jax
import jax.numpy as jnp
from jax import lax
from jax.experimental import pallas as pl
from jax.experimental.pallas import tpu as pltpu
from jax.experimental.pallas import tpu_sc as plsc

N = 10000
E = 320000
D_IN = 128
D_H = 64
D_O = 2

NC = 2
NS = 16
CHUNK = 128
NBUF = 4
N_CHUNKS = 80
EP = NC * NS * CHUNK * N_CHUNKS
EDGES_PER_TILE = CHUNK * N_CHUNKS

NROW = 10240
ROWS_PER_TILE = NROW // NS
JUNK = N
W2P = 16

_mesh = plsc.VectorSubcoreMesh(
    core_axis_name="c", subcore_axis_name="s", num_cores=NC, num_subcores=NS
)


def _zero_vmem_2d(ref, rows, width):
    def body(r, carry):
        for k in range(width // 16):
            ref[r, pl.ds(k * 16, 16)] = jnp.zeros((16,), jnp.float32)
        return carry
    lax.fori_loop(0, rows, body, 0)


@functools.partial(
    pl.kernel,
    out_type=jax.ShapeDtypeStruct((NC, NROW), jnp.float32),
    mesh=_mesh,
    compiler_params=pltpu.CompilerParams(use_tc_tiling_on_sc=False),
    scratch_types=[
        pltpu.VMEM((N_CHUNKS, CHUNK), jnp.int32),
        pltpu.VMEM((CHUNK,), jnp.float32),
        pltpu.VMEM_SHARED((NROW,), jnp.float32),
        pltpu.SemaphoreType.DMA,
    ],
)
def _deg_kernel(dst_hbm, out_hbm, idx_d, ones_v, deg_sh, sem):
    c = lax.axis_index("c")
    s = lax.axis_index("s")
    tile = c * NS + s
    pltpu.sync_copy(dst_hbm.at[pl.ds(tile * N_CHUNKS, N_CHUNKS)], idx_d)
    for k in range(CHUNK // 16):
        ones_v[pl.ds(k * 16, 16)] = jnp.zeros((16,), jnp.float32)
    for r in range(ROWS_PER_TILE // CHUNK):
        pltpu.sync_copy(ones_v, deg_sh.at[pl.ds(s * ROWS_PER_TILE + r * CHUNK, CHUNK)])
    for k in range(CHUNK // 16):
        ones_v[pl.ds(k * 16, 16)] = jnp.ones((16,), jnp.float32)
    plsc.subcore_barrier()

    def fire(j, carry):
        pltpu.async_copy(ones_v, deg_sh.at[idx_d.at[j]], sem, add=True)
        return carry

    lax.fori_loop(0, N_CHUNKS, fire, 0)

    def drain(j, carry):
        pltpu.make_async_copy(ones_v, deg_sh.at[idx_d.at[j]], sem).wait()
        return carry

    lax.fori_loop(0, N_CHUNKS, drain, 0)
    plsc.subcore_barrier()
    pltpu.sync_copy(
        deg_sh.at[pl.ds(s * ROWS_PER_TILE, ROWS_PER_TILE)],
        out_hbm.at[c, pl.ds(s * ROWS_PER_TILE, ROWS_PER_TILE)],
    )


ROWS_STAGE = N // NS


def _make_spmm(width, overlap, stage_table, nbuf):
    @functools.partial(
        pl.kernel,
        out_type=jax.ShapeDtypeStruct((NC, NROW, width), jnp.float32),
        mesh=_mesh,
        compiler_params=pltpu.CompilerParams(use_tc_tiling_on_sc=False),
        scratch_types=[
            pltpu.VMEM((N_CHUNKS + NBUF, CHUNK), jnp.int32),
            pltpu.VMEM((N_CHUNKS, CHUNK), jnp.int32),
            *[pltpu.VMEM((CHUNK, width), jnp.float32) for _ in range(nbuf)],
            pltpu.VMEM_SHARED((NROW, width), jnp.float32),
            *([pltpu.VMEM_SHARED((N, width), jnp.float32)] if stage_table else []),
            *[pltpu.SemaphoreType.DMA for _ in range(nbuf)],
        ],
    )
    def spmm(src_hbm, dst_hbm, tab_hbm, out_hbm, idx_s, idx_d, *scratch):
        rows = list(scratch[:nbuf])
        acc_sh = scratch[nbuf]
        if stage_table:
            tab_ref = scratch[nbuf + 1]
            gsem = list(scratch[nbuf + 2:])
        else:
            tab_ref = tab_hbm
            gsem = list(scratch[nbuf + 1:])
        c = lax.axis_index("c")
        s = lax.axis_index("s")
        tile = c * NS + s
        pltpu.sync_copy(src_hbm.at[pl.ds(tile * N_CHUNKS, N_CHUNKS)], idx_s.at[pl.ds(0, N_CHUNKS)])
        pltpu.sync_copy(dst_hbm.at[pl.ds(tile * N_CHUNKS, N_CHUNKS)], idx_d)
        if stage_table:
            pltpu.sync_copy(
                tab_hbm.at[pl.ds(s * ROWS_STAGE, ROWS_STAGE)],
                tab_ref.at[pl.ds(s * ROWS_STAGE, ROWS_STAGE)],
            )
        for r in range(NBUF):
            for k in range(CHUNK // 16):
                idx_s[N_CHUNKS + r, pl.ds(k * 16, 16)] = jnp.zeros((16,), jnp.int32)
        _zero_vmem_2d(rows[0], CHUNK, width)
        for r in range(ROWS_PER_TILE // CHUNK):
            pltpu.sync_copy(rows[0], acc_sh.at[pl.ds(s * ROWS_PER_TILE + r * CHUNK, CHUNK)])
        plsc.subcore_barrier()

        if overlap:
            for b in range(nbuf):
                pltpu.async_copy(tab_ref.at[idx_s.at[b]], rows[b], gsem[b])

            def body(jj, carry):
                base = jj * nbuf
                for b in range(nbuf):
                    pltpu.make_async_copy(tab_ref.at[idx_s.at[0]], rows[b], gsem[b]).wait()
                    pltpu.sync_copy(rows[b], acc_sh.at[idx_d.at[base + b]], add=True)
                    pltpu.async_copy(tab_ref.at[idx_s.at[base + nbuf + b]], rows[b], gsem[b])
                return carry

            lax.fori_loop(0, N_CHUNKS // nbuf, body, 0)
            for b in range(nbuf):
                pltpu.make_async_copy(tab_ref.at[idx_s.at[0]], rows[b], gsem[b]).wait()
        else:
            def body(j, carry):
                pltpu.async_copy(tab_ref.at[idx_s.at[j]], rows[0], gsem[0]).wait()
                pltpu.sync_copy(rows[0], acc_sh.at[idx_d.at[j]], add=True)
                return carry

            lax.fori_loop(0, N_CHUNKS, body, 0)
        plsc.subcore_barrier()
        pltpu.sync_copy(
            acc_sh.at[pl.ds(s * ROWS_PER_TILE, ROWS_PER_TILE)],
            out_hbm.at[c, pl.ds(s * ROWS_PER_TILE, ROWS_PER_TILE)],
        )

    return spmm


_spmm_h = _make_spmm(D_H, overlap=True, stage_table=True, nbuf=2)
_spmm_o = _make_spmm(W2P, overlap=True, stage_table=True, nbuf=4)

R_BLK = 1000
GRID = N // R_BLK


def _dinv_of(degt_ref):
    deg = degt_ref[:, 0:1] + degt_ref[:, 1:2] + 1.0
    return lax.rsqrt(deg)


def _pre_body(x_ref, w1_ref, degt_ref, hs_ref):
    dinv = _dinv_of(degt_ref)
    h = jnp.dot(x_ref[...], w1_ref[...], preferred_element_type=jnp.float32)
    hs_ref[...] = h * dinv


def _mid_body(e1_ref, hs_ref, degt_ref, w2_ref, b1_ref, gsp_ref):
    dinv = _dinv_of(degt_ref)
    acc = e1_ref[0] + e1_ref[1]
    z = jnp.maximum(dinv * (acc + hs_ref[...]) + b1_ref[...], 0.0)
    g = jnp.dot(z, w2_ref[...], preferred_element_type=jnp.float32)
    gs = g * dinv
    gsp_ref[...] = jnp.concatenate(
        [gs, jnp.zeros((R_BLK, W2P - D_O), jnp.float32)], axis=1
    )


def _out_body(e2_ref, gsp_ref, degt_ref, b2_ref, o_ref):
    dinv = _dinv_of(degt_ref)
    acc = e2_ref[0] + e2_ref[1]
    o = dinv * (acc[:, 0:D_O] + gsp_ref[:, 0:D_O]) + b2_ref[...]
    m = jnp.max(o, axis=1, keepdims=True)
    lse = m + jnp.log(jnp.sum(jnp.exp(o - m), axis=1, keepdims=True))
    o_ref[...] = o - lse


@jax.jit
def kernel(x, edge_index, W1, b1, W2, b2):
    src = edge_index[0].astype(jnp.int32)
    dst = edge_index[1].astype(jnp.int32)
    pad = EP - E
    src_p = jnp.concatenate([src, jnp.zeros((pad,), jnp.int32)]).reshape(-1, CHUNK)
    dst_p = jnp.concatenate([dst, jnp.full((pad,), JUNK, jnp.int32)]).reshape(-1, CHUNK)

    deg_parts = _deg_kernel(dst_p)
    deg_t = jnp.transpose(deg_parts)

    hs = pl.pallas_call(
        _pre_body,
        grid=(GRID,),
        in_specs=[
            pl.BlockSpec((R_BLK, D_IN), lambda i: (i, 0)),
            pl.BlockSpec((D_IN, D_H), lambda i: (0, 0)),
            pl.BlockSpec((R_BLK, NC), lambda i: (i, 0)),
        ],
        out_specs=pl.BlockSpec((R_BLK, D_H), lambda i: (i, 0)),
        out_shape=jax.ShapeDtypeStruct((N, D_H), jnp.float32),
    )(x, W1, deg_t)

    eacc1 = _spmm_h(src_p, dst_p, hs)

    gsp = pl.pallas_call(
        _mid_body,
        grid=(GRID,),
        in_specs=[
            pl.BlockSpec((NC, R_BLK, D_H), lambda i: (0, i, 0)),
            pl.BlockSpec((R_BLK, D_H), lambda i: (i, 0)),
            pl.BlockSpec((R_BLK, NC), lambda i: (i, 0)),
            pl.BlockSpec((D_H, D_O), lambda i: (0, 0)),
            pl.BlockSpec((1, D_H), lambda i: (0, 0)),
        ],
        out_specs=pl.BlockSpec((R_BLK, W2P), lambda i: (i, 0)),
        out_shape=jax.ShapeDtypeStruct((N, W2P), jnp.float32),
    )(eacc1, hs, deg_t, W2, b1.reshape(1, D_H))

    eacc2 = _spmm_o(src_p, dst_p, gsp)

    out = pl.pallas_call(
        _out_body,
        grid=(GRID,),
        in_specs=[
            pl.BlockSpec((NC, R_BLK, W2P), lambda i: (0, i, 0)),
            pl.BlockSpec((R_BLK, W2P), lambda i: (i, 0)),
            pl.BlockSpec((R_BLK, NC), lambda i: (i, 0)),
            pl.BlockSpec((1, D_O), lambda i: (0, 0)),
        ],
        out_specs=pl.BlockSpec((R_BLK, D_O), lambda i: (i, 0)),
        out_shape=jax.ShapeDtypeStruct((N, D_O), jnp.float32),
    )(eacc2, gsp, deg_t, b2.reshape(1, D_O))

    return out


# speedup vs baseline: 1.0063x; 1.0063x over previous
import functools

import jax
import jax.numpy as jnp
from jax import lax
from jax.experimental import pallas as pl
from jax.experimental.pallas import tpu as pltpu
from jax.experimental.pallas import tpu_sc as plsc

N = 10000
E = 320000
D_IN = 128
D_H = 64
D_O = 2

NC = 2
NS = 16
CHUNK = 128
NBUF = 4
N_CHUNKS = 80
EP = NC * NS * CHUNK * N_CHUNKS
EDGES_PER_TILE = CHUNK * N_CHUNKS

NROW = 10240
ROWS_PER_TILE = NROW // NS
JUNK = N
W2P = 16

_mesh = plsc.VectorSubcoreMesh(
    core_axis_name="c", subcore_axis_name="s", num_cores=NC, num_subcores=NS
)


def _zero_vmem_2d(ref, rows, width):
    def body(r, carry):
        for k in range(width // 16):
            ref[r, pl.ds(k * 16, 16)] = jnp.zeros((16,), jnp.float32)
        return carry
    lax.fori_loop(0, rows, body, 0)


@functools.partial(
    pl.kernel,
    out_type=jax.ShapeDtypeStruct((NC, NROW), jnp.float32),
    mesh=_mesh,
    compiler_params=pltpu.CompilerParams(use_tc_tiling_on_sc=False),
    scratch_types=[
        pltpu.VMEM((N_CHUNKS, CHUNK), jnp.int32),
        pltpu.VMEM((CHUNK,), jnp.float32),
        pltpu.VMEM_SHARED((NROW,), jnp.float32),
        pltpu.SemaphoreType.DMA,
    ],
)
def _deg_kernel(dst_hbm, out_hbm, idx_d, ones_v, deg_sh, sem):
    c = lax.axis_index("c")
    s = lax.axis_index("s")
    tile = c * NS + s
    pltpu.sync_copy(dst_hbm.at[pl.ds(tile * N_CHUNKS, N_CHUNKS)], idx_d)
    for k in range(CHUNK // 16):
        ones_v[pl.ds(k * 16, 16)] = jnp.zeros((16,), jnp.float32)
    for r in range(ROWS_PER_TILE // CHUNK):
        pltpu.sync_copy(ones_v, deg_sh.at[pl.ds(s * ROWS_PER_TILE + r * CHUNK, CHUNK)])
    for k in range(CHUNK // 16):
        ones_v[pl.ds(k * 16, 16)] = jnp.ones((16,), jnp.float32)
    plsc.subcore_barrier()

    def fire(j, carry):
        pltpu.async_copy(ones_v, deg_sh.at[idx_d.at[j]], sem, add=True)
        return carry

    lax.fori_loop(0, N_CHUNKS, fire, 0)

    def drain(j, carry):
        pltpu.make_async_copy(ones_v, deg_sh.at[idx_d.at[j]], sem).wait()
        return carry

    lax.fori_loop(0, N_CHUNKS, drain, 0)
    plsc.subcore_barrier()
    pltpu.sync_copy(
        deg_sh.at[pl.ds(s * ROWS_PER_TILE, ROWS_PER_TILE)],
        out_hbm.at[c, pl.ds(s * ROWS_PER_TILE, ROWS_PER_TILE)],
    )


ROWS_STAGE = N // NS


def _make_spmm(width, overlap, stage_table, nbuf):
    @functools.partial(
        pl.kernel,
        out_type=jax.ShapeDtypeStruct((NC, NROW, width), jnp.float32),
        mesh=_mesh,
        compiler_params=pltpu.CompilerParams(use_tc_tiling_on_sc=False),
        scratch_types=[
            pltpu.VMEM((N_CHUNKS + NBUF, CHUNK), jnp.int32),
            pltpu.VMEM((N_CHUNKS, CHUNK), jnp.int32),
            *[pltpu.VMEM((CHUNK, width), jnp.float32) for _ in range(nbuf)],
            pltpu.VMEM_SHARED((NROW, width), jnp.float32),
            *([pltpu.VMEM_SHARED((N, width), jnp.float32)] if stage_table else []),
            *[pltpu.SemaphoreType.DMA for _ in range(nbuf)],
        ],
    )
    def spmm(src_hbm, dst_hbm, tab_hbm, out_hbm, idx_s, idx_d, *scratch):
        rows = list(scratch[:nbuf])
        acc_sh = scratch[nbuf]
        if stage_table:
            tab_ref = scratch[nbuf + 1]
            gsem = list(scratch[nbuf + 2:])
        else:
            tab_ref = tab_hbm
            gsem = list(scratch[nbuf + 1:])
        c = lax.axis_index("c")
        s = lax.axis_index("s")
        tile = c * NS + s
        pltpu.sync_copy(src_hbm.at[pl.ds(tile * N_CHUNKS, N_CHUNKS)], idx_s.at[pl.ds(0, N_CHUNKS)])
        pltpu.sync_copy(dst_hbm.at[pl.ds(tile * N_CHUNKS, N_CHUNKS)], idx_d)
        if stage_table:
            pltpu.sync_copy(
                tab_hbm.at[pl.ds(s * ROWS_STAGE, ROWS_STAGE)],
                tab_ref.at[pl.ds(s * ROWS_STAGE, ROWS_STAGE)],
            )
        for r in range(NBUF):
            for k in range(CHUNK // 16):
                idx_s[N_CHUNKS + r, pl.ds(k * 16, 16)] = jnp.zeros((16,), jnp.int32)
        _zero_vmem_2d(rows[0], CHUNK, width)
        for r in range(ROWS_PER_TILE // CHUNK):
            pltpu.sync_copy(rows[0], acc_sh.at[pl.ds(s * ROWS_PER_TILE + r * CHUNK, CHUNK)])
        plsc.subcore_barrier()

        if overlap:
            for b in range(nbuf):
                pltpu.async_copy(tab_ref.at[idx_s.at[b]], rows[b], gsem[b])

            def body(jj, carry):
                base = jj * nbuf
                for b in range(nbuf):
                    pltpu.make_async_copy(tab_ref.at[idx_s.at[0]], rows[b], gsem[b]).wait()
                    pltpu.sync_copy(rows[b], acc_sh.at[idx_d.at[base + b]], add=True)
                    pltpu.async_copy(tab_ref.at[idx_s.at[base + nbuf + b]], rows[b], gsem[b])
                return carry

            lax.fori_loop(0, N_CHUNKS // nbuf, body, 0)
            for b in range(nbuf):
                pltpu.make_async_copy(tab_ref.at[idx_s.at[0]], rows[b], gsem[b]).wait()
        else:
            def body(j, carry):
                pltpu.async_copy(tab_ref.at[idx_s.at[j]], rows[0], gsem[0]).wait()
                pltpu.sync_copy(rows[0], acc_sh.at[idx_d.at[j]], add=True)
                return carry

            lax.fori_loop(0, N_CHUNKS, body, 0)
        plsc.subcore_barrier()
        pltpu.sync_copy(
            acc_sh.at[pl.ds(s * ROWS_PER_TILE, ROWS_PER_TILE)],
            out_hbm.at[c, pl.ds(s * ROWS_PER_TILE, ROWS_PER_TILE)],
        )

    return spmm


_spmm_h = _make_spmm(D_H, overlap=True, stage_table=True, nbuf=2)
_spmm_o = _make_spmm(W2P, overlap=True, stage_table=True, nbuf=2)

R_BLK = 1000
GRID = N // R_BLK


def _dinv_of(degt_ref):
    deg = degt_ref[:, 0:1] + degt_ref[:, 1:2] + 1.0
    return lax.rsqrt(deg)


def _pre_body(x_ref, w1_ref, degt_ref, hs_ref):
    dinv = _dinv_of(degt_ref)
    h = jnp.dot(x_ref[...], w1_ref[...], preferred_element_type=jnp.float32)
    hs_ref[...] = h * dinv


def _mid_body(e1_ref, hs_ref, degt_ref, w2_ref, b1_ref, gsp_ref):
    dinv = _dinv_of(degt_ref)
    acc = e1_ref[0] + e1_ref[1]
    z = jnp.maximum(dinv * (acc + hs_ref[...]) + b1_ref[...], 0.0)
    g = jnp.dot(z, w2_ref[...], preferred_element_type=jnp.float32)
    gs = g * dinv
    gsp_ref[...] = jnp.concatenate(
        [gs, jnp.zeros((R_BLK, W2P - D_O), jnp.float32)], axis=1
    )


def _out_body(e2_ref, gsp_ref, degt_ref, b2_ref, o_ref):
    dinv = _dinv_of(degt_ref)
    acc = e2_ref[0] + e2_ref[1]
    o = dinv * (acc[:, 0:D_O] + gsp_ref[:, 0:D_O]) + b2_ref[...]
    m = jnp.max(o, axis=1, keepdims=True)
    lse = m + jnp.log(jnp.sum(jnp.exp(o - m), axis=1, keepdims=True))
    o_ref[...] = o - lse


@jax.jit
def kernel(x, edge_index, W1, b1, W2, b2):
    src = edge_index[0].astype(jnp.int32)
    dst = edge_index[1].astype(jnp.int32)
    pad = EP - E
    src_p = jnp.concatenate([src, jnp.zeros((pad,), jnp.int32)]).reshape(-1, CHUNK)
    dst_p = jnp.concatenate([dst, jnp.full((pad,), JUNK, jnp.int32)]).reshape(-1, CHUNK)

    deg_parts = _deg_kernel(dst_p)
    deg_t = jnp.transpose(deg_parts)

    hs = pl.pallas_call(
        _pre_body,
        grid=(GRID,),
        in_specs=[
            pl.BlockSpec((R_BLK, D_IN), lambda i: (i, 0)),
            pl.BlockSpec((D_IN, D_H), lambda i: (0, 0)),
            pl.BlockSpec((R_BLK, NC), lambda i: (i, 0)),
        ],
        out_specs=pl.BlockSpec((R_BLK, D_H), lambda i: (i, 0)),
        out_shape=jax.ShapeDtypeStruct((N, D_H), jnp.float32),
    )(x, W1, deg_t)

    eacc1 = _spmm_h(src_p, dst_p, hs)

    gsp = pl.pallas_call(
        _mid_body,
        grid=(GRID,),
        in_specs=[
            pl.BlockSpec((NC, R_BLK, D_H), lambda i: (0, i, 0)),
            pl.BlockSpec((R_BLK, D_H), lambda i: (i, 0)),
            pl.BlockSpec((R_BLK, NC), lambda i: (i, 0)),
            pl.BlockSpec((D_H, D_O), lambda i: (0, 0)),
            pl.BlockSpec((1, D_H), lambda i: (0, 0)),
        ],
        out_specs=pl.BlockSpec((R_BLK, W2P), lambda i: (i, 0)),
        out_shape=jax.ShapeDtypeStruct((N, W2P), jnp.float32),
    )(eacc1, hs, deg_t, W2, b1.reshape(1, D_H))

    eacc2 = _spmm_o(src_p, dst_p, gsp)

    out = pl.pallas_call(
        _out_body,
        grid=(GRID,),
        in_specs=[
            pl.BlockSpec((NC, R_BLK, W2P), lambda i: (0, i, 0)),
            pl.BlockSpec((R_BLK, W2P), lambda i: (i, 0)),
            pl.BlockSpec((R_BLK, NC), lambda i: (i, 0)),
            pl.BlockSpec((1, D_O), lambda i: (0, 0)),
        ],
        out_specs=pl.BlockSpec((R_BLK, D_O), lambda i: (i, 0)),
        out_shape=jax.ShapeDtypeStruct((N, D_O), jnp.float32),
    )(eacc2, gsp, deg_t, b2.reshape(1, D_O))

    return out


# speedup vs baseline: 1.1143x; 1.1073x over previous
import functools

import jax
import jax.numpy as jnp
from jax import lax
from jax.experimental import pallas as pl
from jax.experimental.pallas import tpu as pltpu
from jax.experimental.pallas import tpu_sc as plsc

N = 10000
E = 320000
D_IN = 128
D_H = 64
D_O = 2

NC = 2
NS = 16
NW = NC * NS
CHUNK = 128
NCHT = E // CHUNK
BASE_CH = NCHT // NW
REM_CH = NCHT % NW
NBUF = 2
IDX_ROWS = BASE_CH + 1 + NBUF

NROW = 10240
ROWS_PER_TILE = NROW // NS
W2P = 16
ROWS_STAGE = N // NS

_mesh = plsc.VectorSubcoreMesh(
    core_axis_name="c", subcore_axis_name="s", num_cores=NC, num_subcores=NS
)


def _tile_schedule(c, s):
    tile = c * NS + s
    start = tile * BASE_CH + jnp.minimum(tile, REM_CH)
    has_tail = tile < REM_CH
    off = jnp.where(tile == NW - 1, 1, 0)
    return start, has_tail, off


def _zero_vmem_2d(ref, rows, width):
    def body(r, carry):
        for k in range(width // 16):
            ref[r, pl.ds(k * 16, 16)] = jnp.zeros((16,), jnp.float32)
        return carry
    lax.fori_loop(0, rows, body, 0)


@functools.partial(
    pl.kernel,
    out_type=jax.ShapeDtypeStruct((NC, NROW), jnp.float32),
    mesh=_mesh,
    compiler_params=pltpu.CompilerParams(use_tc_tiling_on_sc=False),
    scratch_types=[
        pltpu.VMEM((BASE_CH + 1, CHUNK), jnp.int32),
        pltpu.VMEM((CHUNK,), jnp.float32),
        pltpu.VMEM_SHARED((NROW,), jnp.float32),
        pltpu.SemaphoreType.DMA,
    ],
)
def _deg_kernel(e_hbm, out_hbm, idx_d, ones_v, deg_sh, sem):
    c = lax.axis_index("c")
    s = lax.axis_index("s")
    start, has_tail, off = _tile_schedule(c, s)
    cnt = BASE_CH + has_tail.astype(jnp.int32)
    pltpu.sync_copy(e_hbm.at[1, pl.ds(start - off, BASE_CH + 1)], idx_d)
    for k in range(CHUNK // 16):
        ones_v[pl.ds(k * 16, 16)] = jnp.zeros((16,), jnp.float32)
    for r in range(ROWS_PER_TILE // CHUNK):
        pltpu.sync_copy(ones_v, deg_sh.at[pl.ds(s * ROWS_PER_TILE + r * CHUNK, CHUNK)])
    for k in range(CHUNK // 16):
        ones_v[pl.ds(k * 16, 16)] = jnp.ones((16,), jnp.float32)
    plsc.subcore_barrier()

    def fire(j, carry):
        pltpu.async_copy(ones_v, deg_sh.at[idx_d.at[j + off]], sem, add=True)
        return carry

    lax.fori_loop(0, cnt, fire, 0)

    def drain(j, carry):
        pltpu.make_async_copy(ones_v, deg_sh.at[idx_d.at[0]], sem).wait()
        return carry

    lax.fori_loop(0, cnt, drain, 0)
    plsc.subcore_barrier()
    pltpu.sync_copy(
        deg_sh.at[pl.ds(s * ROWS_PER_TILE, ROWS_PER_TILE)],
        out_hbm.at[c, pl.ds(s * ROWS_PER_TILE, ROWS_PER_TILE)],
    )


def _make_spmm(width):
    @functools.partial(
        pl.kernel,
        out_type=jax.ShapeDtypeStruct((NC, NROW, width), jnp.float32),
        mesh=_mesh,
        compiler_params=pltpu.CompilerParams(use_tc_tiling_on_sc=False),
        scratch_types=[
            pltpu.VMEM((IDX_ROWS, CHUNK), jnp.int32),
            pltpu.VMEM((BASE_CH + 1, CHUNK), jnp.int32),
            *[pltpu.VMEM((CHUNK, width), jnp.float32) for _ in range(NBUF)],
            pltpu.VMEM_SHARED((NROW, width), jnp.float32),
            pltpu.VMEM_SHARED((N, width), jnp.float32),
            *[pltpu.SemaphoreType.DMA for _ in range(NBUF)],
        ],
    )
    def spmm(e_hbm, tab_hbm, out_hbm, idx_s, idx_d, *scratch):
        rows = list(scratch[:NBUF])
        acc_sh = scratch[NBUF]
        tab_ref = scratch[NBUF + 1]
        gsem = list(scratch[NBUF + 2:])
        c = lax.axis_index("c")
        s = lax.axis_index("s")
        start, has_tail, off = _tile_schedule(c, s)
        pltpu.sync_copy(
            e_hbm.at[0, pl.ds(start - off, BASE_CH + 1)],
            idx_s.at[pl.ds(0, BASE_CH + 1)],
        )
        pltpu.sync_copy(e_hbm.at[1, pl.ds(start - off, BASE_CH + 1)], idx_d)
        pltpu.sync_copy(
            tab_hbm.at[pl.ds(s * ROWS_STAGE, ROWS_STAGE)],
            tab_ref.at[pl.ds(s * ROWS_STAGE, ROWS_STAGE)],
        )
        for r in range(NBUF):
            for k in range(CHUNK // 16):
                idx_s[BASE_CH + 1 + r, pl.ds(k * 16, 16)] = jnp.zeros((16,), jnp.int32)
        _zero_vmem_2d(rows[0], CHUNK, width)
        for r in range(ROWS_PER_TILE // CHUNK):
            pltpu.sync_copy(rows[0], acc_sh.at[pl.ds(s * ROWS_PER_TILE + r * CHUNK, CHUNK)])
        plsc.subcore_barrier()

        for b in range(NBUF):
            pltpu.async_copy(tab_ref.at[idx_s.at[b + off]], rows[b], gsem[b])

        def body(jj, carry):
            base = jj * NBUF
            for b in range(NBUF):
                pltpu.make_async_copy(tab_ref.at[idx_s.at[0]], rows[b], gsem[b]).wait()
                pltpu.sync_copy(rows[b], acc_sh.at[idx_d.at[base + b + off]], add=True)
                pltpu.async_copy(
                    tab_ref.at[idx_s.at[base + NBUF + b + off]], rows[b], gsem[b]
                )
            return carry

        lax.fori_loop(0, BASE_CH // NBUF, body, 0)
        for b in range(NBUF):
            pltpu.make_async_copy(tab_ref.at[idx_s.at[0]], rows[b], gsem[b]).wait()

        @pl.when(has_tail)
        def _():
            pltpu.async_copy(tab_ref.at[idx_s.at[BASE_CH]], rows[0], gsem[0]).wait()
            pltpu.sync_copy(rows[0], acc_sh.at[idx_d.at[BASE_CH]], add=True)

        plsc.subcore_barrier()
        pltpu.sync_copy(
            acc_sh.at[pl.ds(s * ROWS_PER_TILE, ROWS_PER_TILE)],
            out_hbm.at[c, pl.ds(s * ROWS_PER_TILE, ROWS_PER_TILE)],
        )

    return spmm


_spmm_h = _make_spmm(D_H)
_spmm_o = _make_spmm(W2P)

R_BLK = 1000
GRID = N // R_BLK


def _dinv_of(degt_ref):
    deg = degt_ref[:, 0:1] + degt_ref[:, 1:2] + 1.0
    return lax.rsqrt(deg)


def _pre_body(x_ref, w1_ref, degt_ref, hs_ref):
    dinv = _dinv_of(degt_ref)
    h = jnp.dot(x_ref[...], w1_ref[...], preferred_element_type=jnp.float32)
    hs_ref[...] = h * dinv


def _mid_body(e1_ref, hs_ref, degt_ref, w2_ref, b1_ref, gsp_ref):
    dinv = _dinv_of(degt_ref)
    acc = e1_ref[0] + e1_ref[1]
    z = jnp.maximum(dinv * (acc + hs_ref[...]) + b1_ref[...], 0.0)
    g = jnp.dot(z, w2_ref[...], preferred_element_type=jnp.float32)
    gs = g * dinv
    gsp_ref[...] = jnp.concatenate(
        [gs, jnp.zeros((R_BLK, W2P - D_O), jnp.float32)], axis=1
    )


def _out_body(e2_ref, gsp_ref, degt_ref, b2_ref, o_ref):
    dinv = _dinv_of(degt_ref)
    acc = e2_ref[0] + e2_ref[1]
    o = dinv * (acc[:, 0:D_O] + gsp_ref[:, 0:D_O]) + b2_ref[...]
    m = jnp.max(o, axis=1, keepdims=True)
    lse = m + jnp.log(jnp.sum(jnp.exp(o - m), axis=1, keepdims=True))
    o_ref[...] = o - lse


@jax.jit
def kernel(x, edge_index, W1, b1, W2, b2):
    e32 = edge_index.astype(jnp.int32).reshape(2, NCHT, CHUNK)

    deg_parts = _deg_kernel(e32)
    deg_t = jnp.transpose(deg_parts)

    hs = pl.pallas_call(
        _pre_body,
        grid=(GRID,),
        in_specs=[
            pl.BlockSpec((R_BLK, D_IN), lambda i: (i, 0)),
            pl.BlockSpec((D_IN, D_H), lambda i: (0, 0)),
            pl.BlockSpec((R_BLK, NC), lambda i: (i, 0)),
        ],
        out_specs=pl.BlockSpec((R_BLK, D_H), lambda i: (i, 0)),
        out_shape=jax.ShapeDtypeStruct((N, D_H), jnp.float32),
    )(x, W1, deg_t)

    eacc1 = _spmm_h(e32, hs)

    gsp = pl.pallas_call(
        _mid_body,
        grid=(GRID,),
        in_specs=[
            pl.BlockSpec((NC, R_BLK, D_H), lambda i: (0, i, 0)),
            pl.BlockSpec((R_BLK, D_H), lambda i: (i, 0)),
            pl.BlockSpec((R_BLK, NC), lambda i: (i, 0)),
            pl.BlockSpec((D_H, D_O), lambda i: (0, 0)),
            pl.BlockSpec((1, D_H), lambda i: (0, 0)),
        ],
        out_specs=pl.BlockSpec((R_BLK, W2P), lambda i: (i, 0)),
        out_shape=jax.ShapeDtypeStruct((N, W2P), jnp.float32),
    )(eacc1, hs, deg_t, W2, b1.reshape(1, D_H))

    eacc2 = _spmm_o(e32, gsp)

    out = pl.pallas_call(
        _out_body,
        grid=(GRID,),
        in_specs=[
            pl.BlockSpec((NC, R_BLK, W2P), lambda i: (0, i, 0)),
            pl.BlockSpec((R_BLK, W2P), lambda i: (i, 0)),
            pl.BlockSpec((R_BLK, NC), lambda i: (i, 0)),
            pl.BlockSpec((1, D_O), lambda i: (0, 0)),
        ],
        out_specs=pl.BlockSpec((R_BLK, D_O), lambda i: (i, 0)),
        out_shape=jax.ShapeDtypeStruct((N, D_O), jnp.float32),
    )(eacc2, gsp, deg_t, b2.reshape(1, D_O))

    return out


# speedup vs baseline: 1.1484x; 1.0307x over previous
import functools

import jax
import jax.numpy as jnp
from jax import lax
from jax.experimental import pallas as pl
from jax.experimental.pallas import tpu as pltpu
from jax.experimental.pallas import tpu_sc as plsc

N = 10000
E = 320000
D_IN = 128
D_H = 64
D_O = 2

NC = 2
NS = 16
NW = NC * NS
CHUNK = 128
NCHT = E // CHUNK
BASE_CH = NCHT // NW
REM_CH = NCHT % NW
NBUF = 2
IDX_ROWS = BASE_CH + 1 + NBUF

NROW = 10240
ROWS_PER_TILE = NROW // NS
W2P = 16
ROWS_STAGE = N // NS

_mesh = plsc.VectorSubcoreMesh(
    core_axis_name="c", subcore_axis_name="s", num_cores=NC, num_subcores=NS
)


def _tile_schedule(c, s):
    tile = c * NS + s
    start = tile * BASE_CH + jnp.minimum(tile, REM_CH)
    has_tail = tile < REM_CH
    off = jnp.where(tile == NW - 1, 1, 0)
    return start, has_tail, off


def _zero_vmem_2d(ref, rows, width):
    def body(r, carry):
        for k in range(width // 16):
            ref[r, pl.ds(k * 16, 16)] = jnp.zeros((16,), jnp.float32)
        return carry
    lax.fori_loop(0, rows, body, 0)


@functools.partial(
    pl.kernel,
    out_type=jax.ShapeDtypeStruct((NC, NROW), jnp.float32),
    mesh=_mesh,
    compiler_params=pltpu.CompilerParams(use_tc_tiling_on_sc=False),
    scratch_types=[
        pltpu.VMEM((BASE_CH + 1, CHUNK), jnp.int32),
        pltpu.VMEM((CHUNK,), jnp.float32),
        pltpu.VMEM_SHARED((NROW,), jnp.float32),
        pltpu.SemaphoreType.DMA,
    ],
)
def _deg_kernel(e_hbm, out_hbm, idx_d, ones_v, deg_sh, sem):
    c = lax.axis_index("c")
    s = lax.axis_index("s")
    start, has_tail, off = _tile_schedule(c, s)
    cnt = BASE_CH + has_tail.astype(jnp.int32)
    pltpu.sync_copy(e_hbm.at[1, pl.ds(start - off, BASE_CH + 1)], idx_d)
    for k in range(CHUNK // 16):
        ones_v[pl.ds(k * 16, 16)] = jnp.zeros((16,), jnp.float32)
    for r in range(ROWS_PER_TILE // CHUNK):
        pltpu.sync_copy(ones_v, deg_sh.at[pl.ds(s * ROWS_PER_TILE + r * CHUNK, CHUNK)])
    for k in range(CHUNK // 16):
        ones_v[pl.ds(k * 16, 16)] = jnp.ones((16,), jnp.float32)
    plsc.subcore_barrier()

    def fire(j, carry):
        pltpu.async_copy(ones_v, deg_sh.at[idx_d.at[j + off]], sem, add=True)
        return carry

    lax.fori_loop(0, cnt, fire, 0)

    def drain(j, carry):
        pltpu.make_async_copy(ones_v, deg_sh.at[idx_d.at[0]], sem).wait()
        return carry

    lax.fori_loop(0, cnt, drain, 0)
    plsc.subcore_barrier()
    pltpu.sync_copy(
        deg_sh.at[pl.ds(s * ROWS_PER_TILE, ROWS_PER_TILE)],
        out_hbm.at[c, pl.ds(s * ROWS_PER_TILE, ROWS_PER_TILE)],
    )


def _make_spmm(width):
    @functools.partial(
        pl.kernel,
        out_type=jax.ShapeDtypeStruct((NC, NROW, width), jnp.float32),
        mesh=_mesh,
        compiler_params=pltpu.CompilerParams(use_tc_tiling_on_sc=False),
        scratch_types=[
            pltpu.VMEM((IDX_ROWS, CHUNK), jnp.int32),
            pltpu.VMEM((BASE_CH + 1, CHUNK), jnp.int32),
            *[pltpu.VMEM((CHUNK, width), jnp.float32) for _ in range(NBUF)],
            pltpu.VMEM_SHARED((NROW, width), jnp.float32),
            pltpu.VMEM_SHARED((N, width), jnp.float32),
            *[pltpu.SemaphoreType.DMA for _ in range(NBUF)],
        ],
    )
    def spmm(e_hbm, tab_hbm, out_hbm, idx_s, idx_d, *scratch):
        rows = list(scratch[:NBUF])
        acc_sh = scratch[NBUF]
        tab_ref = scratch[NBUF + 1]
        gsem = list(scratch[NBUF + 2:])
        c = lax.axis_index("c")
        s = lax.axis_index("s")
        start, has_tail, off = _tile_schedule(c, s)
        pltpu.sync_copy(
            e_hbm.at[0, pl.ds(start - off, BASE_CH + 1)],
            idx_s.at[pl.ds(0, BASE_CH + 1)],
        )
        pltpu.sync_copy(e_hbm.at[1, pl.ds(start - off, BASE_CH + 1)], idx_d)
        pltpu.sync_copy(
            tab_hbm.at[pl.ds(s * ROWS_STAGE, ROWS_STAGE)],
            tab_ref.at[pl.ds(s * ROWS_STAGE, ROWS_STAGE)],
        )
        for r in range(NBUF):
            for k in range(CHUNK // 16):
                idx_s[BASE_CH + 1 + r, pl.ds(k * 16, 16)] = jnp.zeros((16,), jnp.int32)
        _zero_vmem_2d(rows[0], CHUNK, width)
        for r in range(ROWS_PER_TILE // CHUNK):
            pltpu.sync_copy(rows[0], acc_sh.at[pl.ds(s * ROWS_PER_TILE + r * CHUNK, CHUNK)])
        plsc.subcore_barrier()

        for b in range(NBUF):
            pltpu.async_copy(tab_ref.at[idx_s.at[b + off]], rows[b], gsem[b])

        def body(jj, carry):
            base = jj * NBUF
            for b in range(NBUF):
                pltpu.make_async_copy(tab_ref.at[idx_s.at[0]], rows[b], gsem[b]).wait()
                pltpu.sync_copy(rows[b], acc_sh.at[idx_d.at[base + b + off]], add=True)
                pltpu.async_copy(
                    tab_ref.at[idx_s.at[base + NBUF + b + off]], rows[b], gsem[b]
                )
            return carry

        lax.fori_loop(0, BASE_CH // NBUF, body, 0)
        for b in range(NBUF):
            pltpu.make_async_copy(tab_ref.at[idx_s.at[0]], rows[b], gsem[b]).wait()

        @pl.when(has_tail)
        def _():
            pltpu.async_copy(tab_ref.at[idx_s.at[BASE_CH]], rows[0], gsem[0]).wait()
            pltpu.sync_copy(rows[0], acc_sh.at[idx_d.at[BASE_CH]], add=True)

        plsc.subcore_barrier()
        pltpu.sync_copy(
            acc_sh.at[pl.ds(s * ROWS_PER_TILE, ROWS_PER_TILE)],
            out_hbm.at[c, pl.ds(s * ROWS_PER_TILE, ROWS_PER_TILE)],
        )

    return spmm


_spmm_h = _make_spmm(D_H)
_spmm_o = _make_spmm(W2P)

R_BLK = 2000
GRID = N // R_BLK


def _dinv_of(degt_ref):
    deg = degt_ref[:, 0:1] + degt_ref[:, 1:2] + 1.0
    return lax.rsqrt(deg)


def _pre_body(x_ref, w1_ref, degt_ref, hs_ref):
    dinv = _dinv_of(degt_ref)
    h = jnp.dot(x_ref[...], w1_ref[...], preferred_element_type=jnp.float32)
    hs_ref[...] = h * dinv


def _mid_body(e1_ref, hs_ref, degt_ref, w2_ref, b1_ref, gsp_ref):
    dinv = _dinv_of(degt_ref)
    acc = e1_ref[0] + e1_ref[1]
    z = jnp.maximum(dinv * (acc + hs_ref[...]) + b1_ref[...], 0.0)
    g = jnp.dot(z, w2_ref[...], preferred_element_type=jnp.float32)
    gs = g * dinv
    gsp_ref[...] = jnp.concatenate(
        [gs, jnp.zeros((R_BLK, W2P - D_O), jnp.float32)], axis=1
    )


def _out_body(e2_ref, gsp_ref, degt_ref, b2_ref, o_ref):
    dinv = _dinv_of(degt_ref)
    acc = e2_ref[0] + e2_ref[1]
    o = dinv * (acc[:, 0:D_O] + gsp_ref[:, 0:D_O]) + b2_ref[...]
    m = jnp.max(o, axis=1, keepdims=True)
    lse = m + jnp.log(jnp.sum(jnp.exp(o - m), axis=1, keepdims=True))
    o_ref[...] = o - lse


@jax.jit
def kernel(x, edge_index, W1, b1, W2, b2):
    e32 = edge_index.astype(jnp.int32).reshape(2, NCHT, CHUNK)

    deg_parts = _deg_kernel(e32)
    deg_t = jnp.transpose(deg_parts)

    hs = pl.pallas_call(
        _pre_body,
        grid=(GRID,),
        in_specs=[
            pl.BlockSpec((R_BLK, D_IN), lambda i: (i, 0)),
            pl.BlockSpec((D_IN, D_H), lambda i: (0, 0)),
            pl.BlockSpec((R_BLK, NC), lambda i: (i, 0)),
        ],
        out_specs=pl.BlockSpec((R_BLK, D_H), lambda i: (i, 0)),
        out_shape=jax.ShapeDtypeStruct((N, D_H), jnp.float32),
    )(x, W1, deg_t)

    eacc1 = _spmm_h(e32, hs)

    gsp = pl.pallas_call(
        _mid_body,
        grid=(GRID,),
        in_specs=[
            pl.BlockSpec((NC, R_BLK, D_H), lambda i: (0, i, 0)),
            pl.BlockSpec((R_BLK, D_H), lambda i: (i, 0)),
            pl.BlockSpec((R_BLK, NC), lambda i: (i, 0)),
            pl.BlockSpec((D_H, D_O), lambda i: (0, 0)),
            pl.BlockSpec((1, D_H), lambda i: (0, 0)),
        ],
        out_specs=pl.BlockSpec((R_BLK, W2P), lambda i: (i, 0)),
        out_shape=jax.ShapeDtypeStruct((N, W2P), jnp.float32),
    )(eacc1, hs, deg_t, W2, b1.reshape(1, D_H))

    eacc2 = _spmm_o(e32, gsp)

    out = pl.pallas_call(
        _out_body,
        grid=(GRID,),
        in_specs=[
            pl.BlockSpec((NC, R_BLK, W2P), lambda i: (0, i, 0)),
            pl.BlockSpec((R_BLK, W2P), lambda i: (i, 0)),
            pl.BlockSpec((R_BLK, NC), lambda i: (i, 0)),
            pl.BlockSpec((1, D_O), lambda i: (0, 0)),
        ],
        out_specs=pl.BlockSpec((R_BLK, D_O), lambda i: (i, 0)),
        out_shape=jax.ShapeDtypeStruct((N, D_O), jnp.float32),
    )(eacc2, gsp, deg_t, b2.reshape(1, D_O))

    return out


# speedup vs baseline: 1.1841x; 1.0311x over previous
import functools

import jax
import jax.numpy as jnp
from jax import lax
from jax.experimental import pallas as pl
from jax.experimental.pallas import tpu as pltpu
from jax.experimental.pallas import tpu_sc as plsc

N = 10000
E = 320000
D_IN = 128
D_H = 64
D_O = 2

NC = 2
NS = 16
NW = NC * NS
CHUNK = 128
NCHT = E // CHUNK
BASE_CH = NCHT // NW
REM_CH = NCHT % NW
NBUF = 2
IDX_ROWS = BASE_CH + 1 + NBUF

NROW = 10240
ROWS_PER_TILE = NROW // NS
W2P = 8
ROWS_STAGE = N // NS

_mesh = plsc.VectorSubcoreMesh(
    core_axis_name="c", subcore_axis_name="s", num_cores=NC, num_subcores=NS
)


def _tile_schedule(c, s):
    tile = c * NS + s
    start = tile * BASE_CH + jnp.minimum(tile, REM_CH)
    has_tail = tile < REM_CH
    off = jnp.where(tile == NW - 1, 1, 0)
    return start, has_tail, off


def _zero_vmem_2d(ref, rows, width):
    def body(r, carry):
        for k in range(width // 16):
            ref[r, pl.ds(k * 16, 16)] = jnp.zeros((16,), jnp.float32)
        return carry
    lax.fori_loop(0, rows, body, 0)


@functools.partial(
    pl.kernel,
    out_type=jax.ShapeDtypeStruct((NC, NROW), jnp.float32),
    mesh=_mesh,
    compiler_params=pltpu.CompilerParams(use_tc_tiling_on_sc=False),
    scratch_types=[
        pltpu.VMEM((BASE_CH + 1, CHUNK), jnp.int32),
        pltpu.VMEM((CHUNK,), jnp.float32),
        pltpu.VMEM_SHARED((NROW,), jnp.float32),
        pltpu.SemaphoreType.DMA,
    ],
)
def _deg_kernel(e_hbm, out_hbm, idx_d, ones_v, deg_sh, sem):
    c = lax.axis_index("c")
    s = lax.axis_index("s")
    start, has_tail, off = _tile_schedule(c, s)
    cnt = BASE_CH + has_tail.astype(jnp.int32)
    pltpu.sync_copy(e_hbm.at[1, pl.ds(start - off, BASE_CH + 1)], idx_d)
    for k in range(CHUNK // 16):
        ones_v[pl.ds(k * 16, 16)] = jnp.zeros((16,), jnp.float32)
    for r in range(ROWS_PER_TILE // CHUNK):
        pltpu.sync_copy(ones_v, deg_sh.at[pl.ds(s * ROWS_PER_TILE + r * CHUNK, CHUNK)])
    for k in range(CHUNK // 16):
        ones_v[pl.ds(k * 16, 16)] = jnp.ones((16,), jnp.float32)
    plsc.subcore_barrier()

    def fire(j, carry):
        pltpu.async_copy(ones_v, deg_sh.at[idx_d.at[j + off]], sem, add=True)
        return carry

    lax.fori_loop(0, cnt, fire, 0)

    def drain(j, carry):
        pltpu.make_async_copy(ones_v, deg_sh.at[idx_d.at[0]], sem).wait()
        return carry

    lax.fori_loop(0, cnt, drain, 0)
    plsc.subcore_barrier()
    pltpu.sync_copy(
        deg_sh.at[pl.ds(s * ROWS_PER_TILE, ROWS_PER_TILE)],
        out_hbm.at[c, pl.ds(s * ROWS_PER_TILE, ROWS_PER_TILE)],
    )


def _make_spmm(width):
    @functools.partial(
        pl.kernel,
        out_type=jax.ShapeDtypeStruct((NC, NROW, width), jnp.float32),
        mesh=_mesh,
        compiler_params=pltpu.CompilerParams(use_tc_tiling_on_sc=False),
        scratch_types=[
            pltpu.VMEM((IDX_ROWS, CHUNK), jnp.int32),
            pltpu.VMEM((BASE_CH + 1, CHUNK), jnp.int32),
            *[pltpu.VMEM((CHUNK, width), jnp.float32) for _ in range(NBUF)],
            pltpu.VMEM_SHARED((NROW, width), jnp.float32),
            pltpu.VMEM_SHARED((N, width), jnp.float32),
            *[pltpu.SemaphoreType.DMA for _ in range(NBUF)],
        ],
    )
    def spmm(e_hbm, tab_hbm, out_hbm, idx_s, idx_d, *scratch):
        rows = list(scratch[:NBUF])
        acc_sh = scratch[NBUF]
        tab_ref = scratch[NBUF + 1]
        gsem = list(scratch[NBUF + 2:])
        c = lax.axis_index("c")
        s = lax.axis_index("s")
        start, has_tail, off = _tile_schedule(c, s)
        pltpu.sync_copy(
            e_hbm.at[0, pl.ds(start - off, BASE_CH + 1)],
            idx_s.at[pl.ds(0, BASE_CH + 1)],
        )
        pltpu.sync_copy(e_hbm.at[1, pl.ds(start - off, BASE_CH + 1)], idx_d)
        pltpu.sync_copy(
            tab_hbm.at[pl.ds(s * ROWS_STAGE, ROWS_STAGE)],
            tab_ref.at[pl.ds(s * ROWS_STAGE, ROWS_STAGE)],
        )
        for r in range(NBUF):
            for k in range(CHUNK // 16):
                idx_s[BASE_CH + 1 + r, pl.ds(k * 16, 16)] = jnp.zeros((16,), jnp.int32)
        _zero_vmem_2d(rows[0], CHUNK, width)
        for r in range(ROWS_PER_TILE // CHUNK):
            pltpu.sync_copy(rows[0], acc_sh.at[pl.ds(s * ROWS_PER_TILE + r * CHUNK, CHUNK)])
        plsc.subcore_barrier()

        for b in range(NBUF):
            pltpu.async_copy(tab_ref.at[idx_s.at[b + off]], rows[b], gsem[b])

        def body(jj, carry):
            base = jj * NBUF
            for b in range(NBUF):
                pltpu.make_async_copy(tab_ref.at[idx_s.at[0]], rows[b], gsem[b]).wait()
                pltpu.sync_copy(rows[b], acc_sh.at[idx_d.at[base + b + off]], add=True)
                pltpu.async_copy(
                    tab_ref.at[idx_s.at[base + NBUF + b + off]], rows[b], gsem[b]
                )
            return carry

        lax.fori_loop(0, BASE_CH // NBUF, body, 0)
        for b in range(NBUF):
            pltpu.make_async_copy(tab_ref.at[idx_s.at[0]], rows[b], gsem[b]).wait()

        @pl.when(has_tail)
        def _():
            pltpu.async_copy(tab_ref.at[idx_s.at[BASE_CH]], rows[0], gsem[0]).wait()
            pltpu.sync_copy(rows[0], acc_sh.at[idx_d.at[BASE_CH]], add=True)

        plsc.subcore_barrier()
        pltpu.sync_copy(
            acc_sh.at[pl.ds(s * ROWS_PER_TILE, ROWS_PER_TILE)],
            out_hbm.at[c, pl.ds(s * ROWS_PER_TILE, ROWS_PER_TILE)],
        )

    return spmm


_spmm_h = _make_spmm(D_H)
_spmm_o = _make_spmm(W2P)

R_BLK = 2000
GRID = N // R_BLK


def _dinv_of(degt_ref):
    deg = degt_ref[:, 0:1] + degt_ref[:, 1:2] + 1.0
    return lax.rsqrt(deg)


def _pre_body(x_ref, w1_ref, degt_ref, hs_ref):
    dinv = _dinv_of(degt_ref)
    h = jnp.dot(x_ref[...], w1_ref[...], preferred_element_type=jnp.float32)
    hs_ref[...] = h * dinv


def _mid_body(e1_ref, hs_ref, degt_ref, w2_ref, b1_ref, gsp_ref):
    dinv = _dinv_of(degt_ref)
    acc = e1_ref[0] + e1_ref[1]
    z = jnp.maximum(dinv * (acc + hs_ref[...]) + b1_ref[...], 0.0)
    g = jnp.dot(z, w2_ref[...], preferred_element_type=jnp.float32)
    gs = g * dinv
    gsp_ref[...] = jnp.concatenate(
        [gs, jnp.zeros((R_BLK, W2P - D_O), jnp.float32)], axis=1
    )


def _out_body(e2_ref, gsp_ref, degt_ref, b2_ref, o_ref):
    dinv = _dinv_of(degt_ref)
    acc = e2_ref[0] + e2_ref[1]
    o = dinv * (acc[:, 0:D_O] + gsp_ref[:, 0:D_O]) + b2_ref[...]
    m = jnp.max(o, axis=1, keepdims=True)
    lse = m + jnp.log(jnp.sum(jnp.exp(o - m), axis=1, keepdims=True))
    o_ref[...] = o - lse


@jax.jit
def kernel(x, edge_index, W1, b1, W2, b2):
    e32 = edge_index.astype(jnp.int32).reshape(2, NCHT, CHUNK)

    deg_parts = _deg_kernel(e32)
    deg_t = jnp.transpose(deg_parts)

    hs = pl.pallas_call(
        _pre_body,
        grid=(GRID,),
        in_specs=[
            pl.BlockSpec((R_BLK, D_IN), lambda i: (i, 0)),
            pl.BlockSpec((D_IN, D_H), lambda i: (0, 0)),
            pl.BlockSpec((R_BLK, NC), lambda i: (i, 0)),
        ],
        out_specs=pl.BlockSpec((R_BLK, D_H), lambda i: (i, 0)),
        out_shape=jax.ShapeDtypeStruct((N, D_H), jnp.float32),
    )(x, W1, deg_t)

    eacc1 = _spmm_h(e32, hs)

    gsp = pl.pallas_call(
        _mid_body,
        grid=(GRID,),
        in_specs=[
            pl.BlockSpec((NC, R_BLK, D_H), lambda i: (0, i, 0)),
            pl.BlockSpec((R_BLK, D_H), lambda i: (i, 0)),
            pl.BlockSpec((R_BLK, NC), lambda i: (i, 0)),
            pl.BlockSpec((D_H, D_O), lambda i: (0, 0)),
            pl.BlockSpec((1, D_H), lambda i: (0, 0)),
        ],
        out_specs=pl.BlockSpec((R_BLK, W2P), lambda i: (i, 0)),
        out_shape=jax.ShapeDtypeStruct((N, W2P), jnp.float32),
    )(eacc1, hs, deg_t, W2, b1.reshape(1, D_H))

    eacc2 = _spmm_o(e32, gsp)

    out = pl.pallas_call(
        _out_body,
        grid=(GRID,),
        in_specs=[
            pl.BlockSpec((NC, R_BLK, W2P), lambda i: (0, i, 0)),
            pl.BlockSpec((R_BLK, W2P), lambda i: (i, 0)),
            pl.BlockSpec((R_BLK, NC), lambda i: (i, 0)),
            pl.BlockSpec((1, D_O), lambda i: (0, 0)),
        ],
        out_specs=pl.BlockSpec((R_BLK, D_O), lambda i: (i, 0)),
        out_shape=jax.ShapeDtypeStruct((N, D_O), jnp.float32),
    )(eacc2, gsp, deg_t, b2.reshape(1, D_O))

    return out
